# baseline (speedup 1.0000x reference)
import jax
import jax.numpy as jnp
from jax.experimental import pallas as pl


def _zero_block(o_ref):
    o_ref[...] = jnp.zeros_like(o_ref)


def kernel(CA, feature_map):
    del CA
    b, c, h, w = feature_map.shape
    total = b * c * h * w
    cols = 512
    rows = total // cols
    grid = 8
    out = pl.pallas_call(
        _zero_block,
        grid=(grid,),
        out_specs=pl.BlockSpec((rows // grid, cols), lambda i: (i, 0)),
        out_shape=jax.ShapeDtypeStruct((rows, cols), feature_map.dtype),
    )()
    return out.reshape(b, c, h, w)


# speedup vs baseline: 3.1676x; 3.1676x over previous
import jax
import jax.numpy as jnp
from jax.experimental import pallas as pl


def _zero_block(o_ref):
    o_ref[...] = jnp.zeros_like(o_ref)


def kernel(CA, feature_map):
    del CA
    b, c, h, w = feature_map.shape
    rows, cols = b * c, h * w
    grid = 8
    out = pl.pallas_call(
        _zero_block,
        grid=(grid,),
        out_specs=pl.BlockSpec((rows // grid, cols), lambda i: (i, 0)),
        out_shape=jax.ShapeDtypeStruct((rows, cols), feature_map.dtype),
    )()
    return out.reshape(b, c, h, w)


# speedup vs baseline: 33.4799x; 10.5696x over previous
import jax
import jax.numpy as jnp
from jax.experimental import pallas as pl


def _zero_block(o_ref):
    o_ref[...] = jnp.zeros_like(o_ref)


def kernel(CA, feature_map):
    del CA
    b, c, h, w = feature_map.shape
    rows, cols = b * h * w, c
    grid = 8
    out = pl.pallas_call(
        _zero_block,
        grid=(grid,),
        out_specs=pl.BlockSpec((rows // grid, cols), lambda i: (i, 0)),
        out_shape=jax.ShapeDtypeStruct((rows, cols), feature_map.dtype),
    )()
    return out.reshape(b, h, w, c).transpose(0, 3, 1, 2)


# speedup vs baseline: 34.4091x; 1.0278x over previous
import jax
import jax.numpy as jnp
from jax.experimental import pallas as pl


def _zero_block(o_ref):
    o_ref[...] = jnp.zeros_like(o_ref)


def kernel(CA, feature_map):
    del CA
    b, c, h, w = feature_map.shape
    rows, cols = b * h * w, c
    grid = 16
    out = pl.pallas_call(
        _zero_block,
        grid=(grid,),
        out_specs=pl.BlockSpec((rows // grid, cols), lambda i: (i, 0)),
        out_shape=jax.ShapeDtypeStruct((rows, cols), feature_map.dtype),
    )()
    return out.reshape(b, h, w, c).transpose(0, 3, 1, 2)
